# baseline (speedup 1.0000x reference)
import functools

import jax
import jax.numpy as jnp
from jax import lax
from jax.experimental import pallas as pl
from jax.experimental.pallas import tpu as pltpu
from jax.experimental.pallas import tpu_sc as plsc


def _make_sc_copy(rows, hidden, dtype):
    info = plsc.get_sparse_core_info()
    nc, ns = info.num_cores, info.num_subcores
    nw = nc * ns
    rows_per_w = rows // nw
    mesh = plsc.VectorSubcoreMesh(core_axis_name="c", subcore_axis_name="s")

    @functools.partial(
        pl.kernel,
        mesh=mesh,
        out_type=jax.ShapeDtypeStruct((rows, hidden), dtype),
        scratch_types=[pltpu.SemaphoreType.DMA],
    )
    def sc_copy(pe_hbm, out_hbm, sem):
        wid = lax.axis_index("s") * nc + lax.axis_index("c")
        base = wid * rows_per_w
        pltpu.async_copy(
            pe_hbm.at[pl.ds(base, rows_per_w)],
            out_hbm.at[pl.ds(base, rows_per_w)],
            sem,
        ).wait()

    return sc_copy


def kernel(x, pe):
    seq_len = x.shape[1]
    hidden = pe.shape[1]
    out = _make_sc_copy(seq_len, hidden, pe.dtype)(pe[:seq_len])
    return out[None, ...]


# speedup vs baseline: 23.0006x; 23.0006x over previous
import functools

import jax
import jax.numpy as jnp
from jax import lax
from jax.experimental import pallas as pl
from jax.experimental.pallas import tpu as pltpu
from jax.experimental.pallas import tpu_sc as plsc

_CHUNK_ROWS = 32


def _make_sc_copy(rows, hidden, dtype):
    info = plsc.get_sparse_core_info()
    nc, ns = info.num_cores, info.num_subcores
    nw = nc * ns
    rows_per_w = rows // nw
    n_chunks = rows_per_w // _CHUNK_ROWS
    mesh = plsc.VectorSubcoreMesh(core_axis_name="c", subcore_axis_name="s")

    @functools.partial(
        pl.kernel,
        mesh=mesh,
        out_type=jax.ShapeDtypeStruct((rows, hidden), dtype),
        scratch_types=[
            pltpu.VMEM((2, _CHUNK_ROWS, hidden), dtype),
            pltpu.SemaphoreType.DMA,
            pltpu.SemaphoreType.DMA,
            pltpu.SemaphoreType.DMA,
            pltpu.SemaphoreType.DMA,
        ],
    )
    def sc_copy(pe_hbm, out_hbm, buf, isem0, isem1, osem0, osem1):
        wid = lax.axis_index("s") * nc + lax.axis_index("c")
        base = wid * rows_per_w
        isem = (isem0, isem1)
        osem = (osem0, osem1)

        def src(c):
            return pe_hbm.at[pl.ds(base + c * _CHUNK_ROWS, _CHUNK_ROWS)]

        def dst(c):
            return out_hbm.at[pl.ds(base + c * _CHUNK_ROWS, _CHUNK_ROWS)]

        ind = {0: pltpu.async_copy(src(0), buf.at[0], isem[0])}
        outd = {}
        for c in range(n_chunks):
            b = c % 2
            ind[c].wait()
            outd[c] = pltpu.async_copy(buf.at[b], dst(c), osem[b])
            if c + 1 < n_chunks:
                if c - 1 >= 0:
                    outd[c - 1].wait()
                ind[c + 1] = pltpu.async_copy(src(c + 1), buf.at[1 - b], isem[1 - b])
        if n_chunks >= 2:
            outd[n_chunks - 2].wait()
        outd[n_chunks - 1].wait()

    return sc_copy


def kernel(x, pe):
    seq_len = x.shape[1]
    hidden = pe.shape[1]
    out = _make_sc_copy(seq_len, hidden, pe.dtype)(pe[:seq_len])
    return out[None, ...]


# speedup vs baseline: 45.4003x; 1.9739x over previous
import jax
import jax.numpy as jnp
from jax.experimental import pallas as pl

_BLOCK_ROWS = 1024


def _copy_body(pe_ref, out_ref):
    out_ref[...] = pe_ref[...]


def kernel(x, pe):
    seq_len = x.shape[1]
    hidden = pe.shape[1]
    grid = (seq_len // _BLOCK_ROWS,)
    out = pl.pallas_call(
        _copy_body,
        grid=grid,
        in_specs=[pl.BlockSpec((_BLOCK_ROWS, hidden), lambda i: (i, 0))],
        out_specs=pl.BlockSpec((_BLOCK_ROWS, hidden), lambda i: (i, 0)),
        out_shape=jax.ShapeDtypeStruct((seq_len, hidden), pe.dtype),
    )(pe)
    return out[None, ...]


# speedup vs baseline: 49.3295x; 1.0865x over previous
import jax
import jax.numpy as jnp
from jax.experimental import pallas as pl

_BLOCK_ROWS = 2048


def _copy_body(pe_ref, out_ref):
    out_ref[...] = pe_ref[...]


def kernel(x, pe):
    seq_len = x.shape[1]
    hidden = pe.shape[1]
    grid = (seq_len // _BLOCK_ROWS,)
    out = pl.pallas_call(
        _copy_body,
        grid=grid,
        in_specs=[pl.BlockSpec((_BLOCK_ROWS, hidden), lambda i: (i, 0))],
        out_specs=pl.BlockSpec((_BLOCK_ROWS, hidden), lambda i: (i, 0)),
        out_shape=jax.ShapeDtypeStruct((seq_len, hidden), pe.dtype),
    )(pe)
    return out[None, ...]
